# baseline (speedup 1.0000x reference)
import functools

import jax
import jax.numpy as jnp
import numpy as np
from jax.experimental import pallas as pl
from jax.experimental.pallas import tpu as pltpu

_NB = {0: [0, 1, 12, 16], 1: [1, 0, 20], 2: [2, 20, 3], 3: [3, 2],
       4: [4, 20, 5], 5: [5, 4, 6], 6: [6, 5, 7], 7: [7, 6, 22],
       8: [8, 20, 9], 9: [9, 8, 10], 10: [10, 9, 11], 11: [11, 10, 24],
       12: [12, 0, 13], 13: [13, 12, 14], 14: [14, 13, 15], 15: [15, 14],
       16: [16, 0, 17], 17: [17, 16, 18], 18: [18, 17, 19], 19: [19, 18],
       20: [20, 1, 2, 4, 8], 21: [21, 22], 22: [22, 21, 7], 23: [23, 24],
       24: [24, 23, 11]}
_E = 8
_D = 32
_T = 9
_HID = 128
_J = 25
_LMAX = 5
_TD = _T * _D


def _moe_kernel(nbidx_ref, ncnt_ref,
                xT_ref, wenc_ref, benc_ref, wmulv_ref, bmulv_ref,
                wd1_ref, bd1_ref, wd2_ref, bd2_ref,
                mu_o, lv_o, xh_o, idx_o,
                h_acc, best_err, best_mu, best_lv, best_xh, best_idx):
    j = pl.program_id(0)
    e = pl.program_id(1)
    B = xT_ref.shape[1]

    cnt = ncnt_ref[j]

    h_acc[...] = jnp.broadcast_to(benc_ref[0, 0], (B, _HID))
    for k in range(_LMAX):
        def _accum(k=k):
            nbk = nbidx_ref[j * _LMAX + k]
            xk = xT_ref[nbk]
            h_acc[...] += jnp.dot(
                xk, wenc_ref[0, 0, k * _TD:(k + 1) * _TD, :],
                preferred_element_type=jnp.float32)
        if k < 2:
            _accum()
        else:
            pl.when(k < cnt)(_accum)
    h = jnp.maximum(h_acc[...], 0.0)

    mulv = jnp.dot(h, wmulv_ref[0, 0], preferred_element_type=jnp.float32)
    mulv = mulv + bmulv_ref[0, 0]
    mu = mulv[:, :_D]
    lv = mulv[:, _D:]

    hd = jnp.dot(mu, wd1_ref[0, 0], preferred_element_type=jnp.float32)
    hd = jnp.maximum(hd + bd1_ref[0, 0], 0.0)
    xh = jnp.dot(hd, wd2_ref[0, 0], preferred_element_type=jnp.float32)
    xh = xh + bd2_ref[0, 0]

    xc = xT_ref[j]
    diff = xh - xc
    err = jnp.mean(diff * diff, axis=-1, keepdims=True)

    @pl.when(e == 0)
    def _():
        best_err[...] = jnp.full((B, 1), jnp.inf, jnp.float32)
        best_idx[...] = jnp.zeros((B, 1), jnp.int32)

    mask = err < best_err[...]
    best_err[...] = jnp.where(mask, err, best_err[...])
    best_mu[...] = jnp.where(mask, mu, best_mu[...])
    best_lv[...] = jnp.where(mask, lv, best_lv[...])
    best_xh[...] = jnp.where(mask, xh, best_xh[...])
    best_idx[...] = jnp.where(mask, e, best_idx[...])

    @pl.when(e == _E - 1)
    def _():
        mu_o[0] = best_mu[...]
        lv_o[0] = best_lv[...]
        xh_o[0] = best_xh[...]
        idx_o[0] = jnp.broadcast_to(best_idx[...], (B, 8))


def _pack_weights(params):
    wenc_js, benc, wmulv, bmulv, wd1, bd1, wd2, bd2 = [], [], [], [], [], [], [], []
    for j in range(_J):
        L = len(_NB[j])
        wj = jnp.stack([params[j][e]['W_enc'] for e in range(_E)])
        wj = wj.reshape(_E, _T, L, _D, _HID).transpose(0, 2, 1, 3, 4)
        wj = wj.reshape(_E, L, _TD, _HID)
        wj = jnp.pad(wj, ((0, 0), (0, _LMAX - L), (0, 0), (0, 0)))
        wenc_js.append(wj.reshape(_E, _LMAX * _TD, _HID))
        benc.append(jnp.stack([params[j][e]['b_enc'] for e in range(_E)]))
        wmulv.append(jnp.stack([
            jnp.concatenate([params[j][e]['W_mu'], params[j][e]['W_lv']], axis=1)
            for e in range(_E)]))
        bmulv.append(jnp.stack([
            jnp.concatenate([params[j][e]['b_mu'], params[j][e]['b_lv']])
            for e in range(_E)]))
        wd1.append(jnp.stack([params[j][e]['W_dec1'] for e in range(_E)]))
        bd1.append(jnp.stack([params[j][e]['b_dec1'] for e in range(_E)]))
        wd2.append(jnp.stack([params[j][e]['W_dec2'] for e in range(_E)]))
        bd2.append(jnp.stack([params[j][e]['b_dec2'] for e in range(_E)]))
    return dict(
        wenc=jnp.stack(wenc_js),
        benc=jnp.stack(benc)[:, :, None, :],
        wmulv=jnp.stack(wmulv),
        bmulv=jnp.stack(bmulv)[:, :, None, :],
        wd1=jnp.stack(wd1),
        bd1=jnp.stack(bd1)[:, :, None, :],
        wd2=jnp.stack(wd2),
        bd2=jnp.stack(bd2)[:, :, None, :],
    )


@functools.partial(jax.jit, static_argnames=())
def _run(x, packed, nbidx, ncnt):
    B = x.shape[0]
    xT = x.transpose(2, 0, 1, 3).reshape(_J, B, _TD)

    def we_map(j, e, *_):
        return (j, e, 0, 0)

    def jo_map(j, e, *_):
        return (j, 0, 0)

    full_x = pl.BlockSpec((_J, B, _TD), lambda j, e, *_: (0, 0, 0))
    per_je = lambda s: pl.BlockSpec((1, 1) + s, we_map)
    per_j = lambda s: pl.BlockSpec((1,) + s, jo_map)

    grid_spec = pltpu.PrefetchScalarGridSpec(
        num_scalar_prefetch=2,
        grid=(_J, _E),
        in_specs=[
            full_x,
            per_je((_LMAX * _TD, _HID)),
            per_je((1, _HID)),
            per_je((_HID, 2 * _D)),
            per_je((1, 2 * _D)),
            per_je((_D, _HID)),
            per_je((1, _HID)),
            per_je((_HID, _TD)),
            per_je((1, _TD)),
        ],
        out_specs=[
            per_j((B, _D)),
            per_j((B, _D)),
            per_j((B, _TD)),
            per_j((B, 8)),
        ],
        scratch_shapes=[
            pltpu.VMEM((B, _HID), jnp.float32),
            pltpu.VMEM((B, 1), jnp.float32),
            pltpu.VMEM((B, _D), jnp.float32),
            pltpu.VMEM((B, _D), jnp.float32),
            pltpu.VMEM((B, _TD), jnp.float32),
            pltpu.VMEM((B, 1), jnp.int32),
        ],
    )

    mu_o, lv_o, xh_o, idx_o = pl.pallas_call(
        _moe_kernel,
        grid_spec=grid_spec,
        out_shape=[
            jax.ShapeDtypeStruct((_J, B, _D), jnp.float32),
            jax.ShapeDtypeStruct((_J, B, _D), jnp.float32),
            jax.ShapeDtypeStruct((_J, B, _TD), jnp.float32),
            jax.ShapeDtypeStruct((_J, B, 8), jnp.int32),
        ],
    )(nbidx, ncnt, xT,
      packed['wenc'], packed['benc'], packed['wmulv'], packed['bmulv'],
      packed['wd1'], packed['bd1'], packed['wd2'], packed['bd2'])

    out_mu = mu_o.transpose(1, 0, 2)
    out_lv = lv_o.transpose(1, 0, 2)
    out_xh = xh_o.reshape(_J, B, _T, _D).transpose(1, 2, 0, 3)
    out_idx = idx_o[:, :, 0].transpose(1, 0)
    return out_mu, out_lv, out_xh, out_idx


def kernel(x, params):
    packed = _pack_weights(params)
    nb = np.zeros((_J, _LMAX), np.int32)
    cnt = np.zeros((_J,), np.int32)
    for j in range(_J):
        L = len(_NB[j])
        nb[j, :L] = _NB[j]
        cnt[j] = L
    return _run(x, packed, jnp.asarray(nb.reshape(-1)), jnp.asarray(cnt))


# speedup vs baseline: 1.0101x; 1.0101x over previous
import jax
import jax.numpy as jnp
import numpy as np
from jax.experimental import pallas as pl
from jax.experimental.pallas import tpu as pltpu

_NB = {0: [0, 1, 12, 16], 1: [1, 0, 20], 2: [2, 20, 3], 3: [3, 2],
       4: [4, 20, 5], 5: [5, 4, 6], 6: [6, 5, 7], 7: [7, 6, 22],
       8: [8, 20, 9], 9: [9, 8, 10], 10: [10, 9, 11], 11: [11, 10, 24],
       12: [12, 0, 13], 13: [13, 12, 14], 14: [14, 13, 15], 15: [15, 14],
       16: [16, 0, 17], 17: [17, 16, 18], 18: [18, 17, 19], 19: [19, 18],
       20: [20, 1, 2, 4, 8], 21: [21, 22], 22: [22, 21, 7], 23: [23, 24],
       24: [24, 23, 11]}
_E = 8
_D = 32
_T = 9
_HID = 128
_J = 25
_LMAX = 5
_TD = _T * _D

_LENS = [len(_NB[j]) for j in range(_J)]
_WOFF = np.cumsum([0] + [_E * L for L in _LENS])[:-1].astype(np.int32)

_TSRC = np.zeros((_J, _LMAX, _T), np.int32)
_JSRC = np.zeros((_J, _LMAX, _T), np.int32)
for _j in range(_J):
    _L = _LENS[_j]
    for _p in range(_LMAX * _T):
        _c, _i = divmod(_p, _T)
        if _p < _T * _L:
            _TSRC[_j, _c, _i] = _p // _L
            _JSRC[_j, _c, _i] = _NB[_j][_p % _L]


def _moe_kernel(ncnt_ref, woff_ref,
                dx_ref, xc_ref, wenc_ref, benc_ref, wmulv_ref, bmulv_ref,
                wd1_ref, bd1_ref, wd2_ref, bd2_ref,
                mu_o, lv_o, xh_o, idx_o,
                h_acc, best_err, best_mu, best_lv, best_xh, best_idx):
    j = pl.program_id(0)
    e = pl.program_id(1)
    c = pl.program_id(2)
    B = dx_ref.shape[2]
    cnt = ncnt_ref[j]

    @pl.when(c == 0)
    def _():
        h_acc[...] = jnp.broadcast_to(benc_ref[0, 0], (B, _HID))

    @pl.when(c < cnt)
    def _():
        h_acc[...] += jnp.dot(dx_ref[0, c], wenc_ref[0],
                              preferred_element_type=jnp.float32)

    @pl.when(c == _LMAX - 1)
    def _():
        h = jnp.maximum(h_acc[...], 0.0)

        mulv = jnp.dot(h, wmulv_ref[0, 0], preferred_element_type=jnp.float32)
        mulv = mulv + bmulv_ref[0, 0]
        mu = mulv[:, :_D]
        lv = mulv[:, _D:]

        hd = jnp.dot(mu, wd1_ref[0, 0], preferred_element_type=jnp.float32)
        hd = jnp.maximum(hd + bd1_ref[0, 0], 0.0)
        xh = jnp.dot(hd, wd2_ref[0, 0], preferred_element_type=jnp.float32)
        xh = xh + bd2_ref[0, 0]

        diff = xh - xc_ref[0]
        err = jnp.mean(diff * diff, axis=-1, keepdims=True)

        @pl.when(e == 0)
        def _():
            best_err[...] = jnp.full((B, 1), jnp.inf, jnp.float32)
            best_idx[...] = jnp.zeros((B, 1), jnp.int32)

        mask = err < best_err[...]
        best_err[...] = jnp.where(mask, err, best_err[...])
        best_mu[...] = jnp.where(mask, mu, best_mu[...])
        best_lv[...] = jnp.where(mask, lv, best_lv[...])
        best_xh[...] = jnp.where(mask, xh, best_xh[...])
        best_idx[...] = jnp.where(mask, e, best_idx[...])

        @pl.when(e == _E - 1)
        def _():
            mu_o[0] = best_mu[...]
            lv_o[0] = best_lv[...]
            xh_o[0] = best_xh[...]
            idx_o[0] = jnp.broadcast_to(best_idx[...], (B, 8))


def _run(x, wenc, benc, wmulv, bmulv, wd1, bd1, wd2, bd2, ncnt, woff):
    B = x.shape[0]
    g = x[:, _TSRC, _JSRC, :]
    dx = g.transpose(1, 2, 0, 3, 4).reshape(_J, _LMAX, B, _TD)
    xT = x.transpose(2, 0, 1, 3).reshape(_J, B, _TD)

    def je_map(j, e, c, *_):
        return (j, e, 0, 0)

    per_je = lambda s: pl.BlockSpec((1, 1) + s, je_map)
    per_j = lambda s: pl.BlockSpec((1,) + s, lambda j, e, c, *_: (j, 0, 0))

    def wenc_map(j, e, c, ncnt_ref, woff_ref):
        L = ncnt_ref[j]
        return (woff_ref[j] + e * L + jnp.minimum(c, L - 1), 0, 0)

    grid_spec = pltpu.PrefetchScalarGridSpec(
        num_scalar_prefetch=2,
        grid=(_J, _E, _LMAX),
        in_specs=[
            pl.BlockSpec((1, _LMAX, B, _TD), lambda j, e, c, *_: (j, 0, 0, 0)),
            per_j((B, _TD)),
            pl.BlockSpec((1, _TD, _HID), wenc_map),
            per_je((1, _HID)),
            per_je((_HID, 2 * _D)),
            per_je((1, 2 * _D)),
            per_je((_D, _HID)),
            per_je((1, _HID)),
            per_je((_HID, _TD)),
            per_je((1, _TD)),
        ],
        out_specs=[
            per_j((B, _D)),
            per_j((B, _D)),
            per_j((B, _TD)),
            per_j((B, 8)),
        ],
        scratch_shapes=[
            pltpu.VMEM((B, _HID), jnp.float32),
            pltpu.VMEM((B, 1), jnp.float32),
            pltpu.VMEM((B, _D), jnp.float32),
            pltpu.VMEM((B, _D), jnp.float32),
            pltpu.VMEM((B, _TD), jnp.float32),
            pltpu.VMEM((B, 1), jnp.int32),
        ],
    )

    mu_o, lv_o, xh_o, idx_o = pl.pallas_call(
        _moe_kernel,
        grid_spec=grid_spec,
        out_shape=[
            jax.ShapeDtypeStruct((_J, B, _D), jnp.float32),
            jax.ShapeDtypeStruct((_J, B, _D), jnp.float32),
            jax.ShapeDtypeStruct((_J, B, _TD), jnp.float32),
            jax.ShapeDtypeStruct((_J, B, 8), jnp.int32),
        ],
    )(ncnt, woff, dx, xT,
      wenc, benc, wmulv, bmulv, wd1, bd1, wd2, bd2)

    out_mu = mu_o.transpose(1, 0, 2)
    out_lv = lv_o.transpose(1, 0, 2)
    out_xh = xh_o.reshape(_J, B, _T, _D).transpose(1, 2, 0, 3)
    out_idx = idx_o[:, :, 0].transpose(1, 0)
    return out_mu, out_lv, out_xh, out_idx


def kernel(x, params):
    flat = [params[j][e] for j in range(_J) for e in range(_E)]
    wenc = jnp.concatenate([p['W_enc'] for p in flat], axis=0)
    wenc = wenc.reshape(-1, _TD, _HID)
    benc = jnp.stack([p['b_enc'] for p in flat]).reshape(_J, _E, 1, _HID)
    wmu = jnp.stack([p['W_mu'] for p in flat])
    wlv = jnp.stack([p['W_lv'] for p in flat])
    wmulv = jnp.concatenate([wmu, wlv], axis=-1).reshape(_J, _E, _HID, 2 * _D)
    bmu = jnp.stack([p['b_mu'] for p in flat])
    blv = jnp.stack([p['b_lv'] for p in flat])
    bmulv = jnp.concatenate([bmu, blv], axis=-1).reshape(_J, _E, 1, 2 * _D)
    wd1 = jnp.stack([p['W_dec1'] for p in flat]).reshape(_J, _E, _D, _HID)
    bd1 = jnp.stack([p['b_dec1'] for p in flat]).reshape(_J, _E, 1, _HID)
    wd2 = jnp.stack([p['W_dec2'] for p in flat]).reshape(_J, _E, _HID, _TD)
    bd2 = jnp.stack([p['b_dec2'] for p in flat]).reshape(_J, _E, 1, _TD)
    return _run(x, wenc, benc, wmulv, bmulv, wd1, bd1, wd2, bd2,
                jnp.asarray(np.array(_LENS, np.int32)), jnp.asarray(_WOFF))
